# Initial kernel scaffold; baseline (speedup 1.0000x reference)
import jax
import jax.numpy as jnp
from jax.experimental import pallas as pl


def kernel(x, edge_index, edge_attr, batch, y, W_gat, a_src, a_dst, b_gat, W1, b1, W2, b2, W3, b3, W_lin, b_lin):
    raise NotImplementedError("write your pallas kernel here")


# speedup vs baseline: 13.1733x; 13.1733x over previous
import functools

import jax
import jax.numpy as jnp
from jax import lax
from jax.experimental import pallas as pl
from jax.experimental.pallas import tpu as pltpu
from jax.experimental.pallas import tpu_sc as plsc

F32 = jnp.float32
I32 = jnp.int32

NC = 2
NS = 16
NW = NC * NS
LANES = 16
K = 80


def _tc_gat_pre(x, W, a_s, a_d):
    N, D = x.shape
    H = W.shape[1]
    R = 1000
    G = N // R

    def body(x_ref, w_ref, as_ref, ad_ref, xw_ref, als_ref, ald_ref):
        m = jnp.dot(x_ref[...], w_ref[...], preferred_element_type=F32)
        xw_ref[...] = m
        als_ref[...] = jnp.dot(m, as_ref[...], preferred_element_type=F32)
        ald_ref[...] = jnp.dot(m, ad_ref[...], preferred_element_type=F32)

    return pl.pallas_call(
        body,
        grid=(G,),
        in_specs=[
            pl.BlockSpec((R, D), lambda i: (i, 0)),
            pl.BlockSpec((D, H), lambda i: (0, 0)),
            pl.BlockSpec((H, 1), lambda i: (0, 0)),
            pl.BlockSpec((H, 1), lambda i: (0, 0)),
        ],
        out_specs=[
            pl.BlockSpec((R, H), lambda i: (i, 0)),
            pl.BlockSpec((R, 1), lambda i: (i, 0)),
            pl.BlockSpec((R, 1), lambda i: (i, 0)),
        ],
        out_shape=[
            jax.ShapeDtypeStruct((N, H), F32),
            jax.ShapeDtypeStruct((N, 1), F32),
            jax.ShapeDtypeStruct((N, 1), F32),
        ],
    )(x, W, a_s.reshape(H, 1), a_d.reshape(H, 1))


def _tc_gat_post(agg, s2, deg2, b_gat, W1):
    _, N, H = agg.shape
    H2 = W1.shape[1]
    R = 1000
    G = N // R

    def body(agg_ref, s2_ref, d2_ref, bg_ref, w1_ref,
             xw1_ref, tab1_ref, dinv_ref):
        aggsum = agg_ref[0] + agg_ref[1]
        sinv = 1.0 / (s2_ref[0] + s2_ref[1] + 1e-16)
        deg = d2_ref[0] + d2_ref[1] + 1.0
        dinv = lax.rsqrt(deg)
        h1 = jnp.tanh(aggsum * sinv + bg_ref[...])
        m = jnp.dot(h1, w1_ref[...], preferred_element_type=F32)
        xw1_ref[...] = m
        tab1_ref[...] = m * dinv
        dinv_ref[...] = dinv

    return pl.pallas_call(
        body,
        grid=(G,),
        in_specs=[
            pl.BlockSpec((2, R, H), lambda i: (0, i, 0)),
            pl.BlockSpec((2, R, 1), lambda i: (0, i, 0)),
            pl.BlockSpec((2, R, 1), lambda i: (0, i, 0)),
            pl.BlockSpec((1, H), lambda i: (0, 0)),
            pl.BlockSpec((H, H2), lambda i: (0, 0)),
        ],
        out_specs=[
            pl.BlockSpec((R, H2), lambda i: (i, 0)),
            pl.BlockSpec((R, H2), lambda i: (i, 0)),
            pl.BlockSpec((R, 1), lambda i: (i, 0)),
        ],
        out_shape=[
            jax.ShapeDtypeStruct((N, H2), F32),
            jax.ShapeDtypeStruct((N, H2), F32),
            jax.ShapeDtypeStruct((N, 1), F32),
        ],
    )(agg, s2.reshape(2, N, 1), deg2.reshape(2, N, 1),
      b_gat.reshape(1, H), W1)


def _tc_gcn_layer(aggs, xwp, dinv, b, W, splits):
    N, dp = xwp.shape
    dq = W.shape[1]
    R = 1000
    G = N // R
    na = len(aggs)

    def body(*refs):
        agg_refs = refs[:na]
        xwp_ref, dinv_ref, b_ref, w_ref = refs[na:na + 4]
        out_refs = refs[na + 4:]
        xwq_ref = out_refs[0]
        tab_refs = out_refs[1:]
        aggsum = jnp.concatenate([a[0] + a[1] for a in agg_refs], axis=1)
        dinv = dinv_ref[...]
        h = jnp.tanh(dinv * aggsum + dinv * dinv * xwp_ref[...] + b_ref[...])
        m = jnp.dot(h, w_ref[...], preferred_element_type=F32)
        xwq_ref[...] = m
        tab = m * dinv
        c0 = 0
        for t_ref, cw in zip(tab_refs, splits):
            t_ref[...] = tab[:, c0:c0 + cw]
            c0 += cw

    in_specs = [pl.BlockSpec((2, R, a.shape[2]), lambda i: (0, i, 0))
                for a in aggs]
    in_specs += [
        pl.BlockSpec((R, dp), lambda i: (i, 0)),
        pl.BlockSpec((R, 1), lambda i: (i, 0)),
        pl.BlockSpec((1, dp), lambda i: (0, 0)),
        pl.BlockSpec((dp, dq), lambda i: (0, 0)),
    ]
    out_specs = [pl.BlockSpec((R, dq), lambda i: (i, 0))]
    out_specs += [pl.BlockSpec((R, cw), lambda i: (i, 0)) for cw in splits]
    out_shape = [jax.ShapeDtypeStruct((N, dq), F32)]
    out_shape += [jax.ShapeDtypeStruct((N, cw), F32) for cw in splits]

    return pl.pallas_call(
        body,
        grid=(G,),
        in_specs=in_specs,
        out_specs=out_specs,
        out_shape=out_shape,
    )(*aggs, xwp, dinv, b.reshape(1, dp), W)


def _tc_pool(aggs, xw3, dinv, b3, batch, nb):
    N, dq = xw3.shape
    R = 1000
    G = N // R
    na = len(aggs)

    def body(*refs):
        agg_refs = refs[:na]
        xw_ref, dinv_ref, b_ref, bat_ref = refs[na:na + 4]
        psum_ref, pcnt_ref = refs[na + 4:]
        aggsum = jnp.concatenate([a[0] + a[1] for a in agg_refs], axis=1)
        dinv = dinv_ref[...]
        h4 = jnp.tanh(dinv * aggsum + dinv * dinv * xw_ref[...] + b_ref[...])
        ids = lax.broadcasted_iota(I32, (nb, R), 0)
        onehot = (ids == bat_ref[...].reshape(1, R)).astype(F32)
        ps = jnp.dot(onehot, h4, preferred_element_type=F32)
        pc = jnp.sum(onehot, axis=1, keepdims=True) * jnp.ones((1, 128), F32)

        @pl.when(pl.program_id(0) == 0)
        def _():
            psum_ref[...] = ps
            pcnt_ref[...] = pc

        @pl.when(pl.program_id(0) != 0)
        def _():
            psum_ref[...] += ps
            pcnt_ref[...] += pc

    in_specs = [pl.BlockSpec((2, R, a.shape[2]), lambda i: (0, i, 0))
                for a in aggs]
    in_specs += [
        pl.BlockSpec((R, dq), lambda i: (i, 0)),
        pl.BlockSpec((R, 1), lambda i: (i, 0)),
        pl.BlockSpec((1, dq), lambda i: (0, 0)),
        pl.BlockSpec((R, 1), lambda i: (i, 0)),
    ]
    return pl.pallas_call(
        body,
        grid=(G,),
        in_specs=in_specs,
        out_specs=[
            pl.BlockSpec((nb, dq), lambda i: (0, 0)),
            pl.BlockSpec((nb, 128), lambda i: (0, 0)),
        ],
        out_shape=[
            jax.ShapeDtypeStruct((nb, dq), F32),
            jax.ShapeDtypeStruct((nb, 128), F32),
        ],
    )(*aggs, xw3, dinv, b3.reshape(1, dq), batch.reshape(N, 1))


def _tc_head(psum, pcnt, W_lin, b_lin):
    nb, dq = psum.shape
    dout = W_lin.shape[1]

    def body(ps_ref, pc_ref, wl_ref, bl_ref, out_ref):
        cnt = jnp.maximum(pc_ref[:, 0:1], 1.0)
        pooled = ps_ref[...] / cnt
        out_ref[...] = jnp.dot(pooled, wl_ref[...],
                               preferred_element_type=F32) + bl_ref[...]

    return pl.pallas_call(
        body,
        out_shape=jax.ShapeDtypeStruct((nb, dout), F32),
    )(psum, pcnt, W_lin, b_lin.reshape(1, dout))


def _sc_scalar_pass(als, ald, src3, dst3, w3, zn):
    N = als.shape[0]
    _, NCH, _ = src3.shape
    E = NW * NCH * K
    mesh = plsc.VectorSubcoreMesh(core_axis_name="c", subcore_axis_name="s",
                                  num_cores=NC, num_subcores=NS)

    def body(als_h, ald_h, src_h, dst_h, w_h, zn_h,
             p_h, s2_h, deg2_h,
             als_v, ald_v, srcv, dstv, wv, pbuf, sacc, dacc, sem):
        c = lax.axis_index("c")
        sid = lax.axis_index("s")
        wid = c * NS + sid
        pltpu.sync_copy(als_h, als_v)
        pltpu.sync_copy(ald_h, ald_v)
        pltpu.sync_copy(src_h.at[wid], srcv)
        pltpu.sync_copy(dst_h.at[wid], dstv)
        pltpu.sync_copy(w_h.at[wid], wv)

        @pl.when(sid == 0)
        def _():
            pltpu.sync_copy(zn_h, sacc)
            pltpu.sync_copy(zn_h, dacc)

        plsc.subcore_barrier()

        def chunk(j, carry):
            for g in range(K // LANES):
                sl = pl.ds(g * LANES, LANES)
                z = (plsc.load_gather(als_v, [srcv[j, sl]])
                     + plsc.load_gather(ald_v, [dstv[j, sl]]))
                z = jnp.where(z > 0, z, 0.2 * z)
                pbuf[sl] = jnp.exp(z)
            pltpu.sync_copy(pbuf, p_h.at[wid, j])
            pltpu.sync_copy(pbuf, sacc.at[dstv.at[j]], add=True)
            pltpu.sync_copy(wv.at[j], dacc.at[dstv.at[j]], add=True)
            return carry

        lax.fori_loop(0, NCH, chunk, 0)
        plsc.subcore_barrier()

        @pl.when(sid == 0)
        def _():
            pltpu.sync_copy(sacc, s2_h.at[c])
            pltpu.sync_copy(dacc, deg2_h.at[c])

    kern = pl.kernel(
        body,
        compiler_params=pltpu.CompilerParams(needs_layout_passes=False,
                                             use_tc_tiling_on_sc=False),
        out_type=[
            jax.ShapeDtypeStruct((NW, NCH, K), F32),
            jax.ShapeDtypeStruct((NC, N), F32),
            jax.ShapeDtypeStruct((NC, N), F32),
        ],
        mesh=mesh,
        scratch_types=[
            pltpu.VMEM((N,), F32),
            pltpu.VMEM((N,), F32),
            pltpu.VMEM((NCH, K), I32),
            pltpu.VMEM((NCH, K), I32),
            pltpu.VMEM((NCH, K), F32),
            pltpu.VMEM((K,), F32),
            pltpu.VMEM_SHARED((N,), F32),
            pltpu.VMEM_SHARED((N,), F32),
            pltpu.SemaphoreType.DMA,
        ],
    )
    p3, s2, deg2 = kern(als, ald, src3, dst3, w3, zn)
    return p3, s2, deg2


def _sc_edge_sum(table, src3, dst3, cf3, z2d):
    N, dc = table.shape
    _, NCH, _ = src3.shape
    mesh = plsc.VectorSubcoreMesh(core_axis_name="c", subcore_axis_name="s",
                                  num_cores=NC, num_subcores=NS)

    def body(tab_h, src_h, dst_h, cf_h, z_h, agg_h,
             srcv, dstv, cv, rows, acc, sem):
        c = lax.axis_index("c")
        sid = lax.axis_index("s")
        wid = c * NS + sid
        pltpu.sync_copy(src_h.at[wid], srcv)
        pltpu.sync_copy(dst_h.at[wid], dstv)
        pltpu.sync_copy(cf_h.at[wid], cv)

        @pl.when(sid == 0)
        def _():
            pltpu.sync_copy(z_h, acc)

        plsc.subcore_barrier()

        def chunk(j, carry):
            pltpu.async_copy(tab_h.at[srcv.at[j]], rows, sem).wait()
            for e in range(K):
                cb = plsc.load_gather(
                    cv, [jnp.full((LANES,), j, I32), jnp.full((LANES,), e, I32)])
                for cc in range(dc // LANES):
                    sl = pl.ds(cc * LANES, LANES)
                    rows[e, sl] = rows[e, sl] * cb
            pltpu.sync_copy(rows, acc.at[dstv.at[j]], add=True)
            return carry

        lax.fori_loop(0, NCH, chunk, 0)
        plsc.subcore_barrier()

        @pl.when(sid == 0)
        def _():
            pltpu.sync_copy(acc, agg_h.at[c])

    kern = pl.kernel(
        body,
        compiler_params=pltpu.CompilerParams(needs_layout_passes=False,
                                             use_tc_tiling_on_sc=False),
        out_type=jax.ShapeDtypeStruct((NC, N, dc), F32),
        mesh=mesh,
        scratch_types=[
            pltpu.VMEM((NCH, K), I32),
            pltpu.VMEM((NCH, K), I32),
            pltpu.VMEM((NCH, K), F32),
            pltpu.VMEM((K, dc), F32),
            pltpu.VMEM_SHARED((N, dc), F32),
            pltpu.SemaphoreType.DMA,
        ],
    )
    return kern(table, src3, dst3, cf3, z2d)


def kernel(x, edge_index, edge_attr, batch, y, W_gat, a_src, a_dst, b_gat,
           W1, b1, W2, b2, W3, b3, W_lin, b_lin):
    N = x.shape[0]
    nb = y.shape[0]
    E = edge_index.shape[1]
    NCH = E // (NW * K)
    src3 = edge_index[0].astype(I32).reshape(NW, NCH, K)
    dst3 = edge_index[1].astype(I32).reshape(NW, NCH, K)
    w3 = edge_attr.reshape(-1).astype(F32).reshape(NW, NCH, K)
    bat = batch.astype(I32)

    zn = jnp.zeros((N,), F32)
    z64 = jnp.zeros((N, 64), F32)
    z128 = jnp.zeros((N, 128), F32)

    xw, als, ald = _tc_gat_pre(x, W_gat, a_src, a_dst)
    p3, s2, deg2 = _sc_scalar_pass(als.reshape(N), ald.reshape(N),
                                   src3, dst3, w3, zn)
    agg_g = _sc_edge_sum(xw, src3, dst3, p3, z64)
    xw1, tab1, dinv = _tc_gat_post(agg_g, s2, deg2, b_gat, W1)

    agg1 = _sc_edge_sum(tab1, src3, dst3, w3, z128)
    xw2, tab2a, tab2b = _tc_gcn_layer([agg1], xw1, dinv, b1, W2, (128, 64))

    agg2a = _sc_edge_sum(tab2a, src3, dst3, w3, z128)
    agg2b = _sc_edge_sum(tab2b, src3, dst3, w3, z64)
    xw3, tab3a, tab3b = _tc_gcn_layer([agg2a, agg2b], xw2, dinv, b2, W3,
                                      (128, 128))

    agg3a = _sc_edge_sum(tab3a, src3, dst3, w3, z128)
    agg3b = _sc_edge_sum(tab3b, src3, dst3, w3, z128)

    psum, pcnt = _tc_pool([agg3a, agg3b], xw3, dinv, b3, bat, nb)
    return _tc_head(psum, pcnt, W_lin, b_lin)
